# baseline (speedup 1.0000x reference)
import jax
import jax.numpy as jnp
from jax import lax
from jax.experimental import pallas as pl
from jax.experimental.pallas import tpu as pltpu
from jax.experimental.pallas import tpu_sc as plsc

MINUTE_SIZE = 60
D_MODEL = 512
_V = 64

_N = 4096 * 200
_K = 563200
_NSC = _N - _K
_BLK = 2048
_GRID = _K // _BLK

_NW = 32
_PER_W = _NSC // _NW
_CHUNK = 80
_STEPS = _PER_W // _CHUNK
_L = 16
_G = _CHUNK // _L
_DRAIN = 64000
_NDRAIN = _PER_W * D_MODEL // _DRAIN


def _tc_body(x_ref, w_ref, o_ref):
    idx = x_ref[:, 1]
    onehot = (idx[:, None] == jax.lax.broadcasted_iota(
        jnp.int32, (1, _V), 1)).astype(jnp.float32)
    o_ref[...] = jnp.dot(onehot, w_ref[...],
                         preferred_element_type=jnp.float32)


def _sc_kernel(x_hbm, w_hbm, out_hbm, w_tile, dummy, xbufs, idxs,
               xsems, wsem):
    wid = lax.axis_index("s") * 2 + lax.axis_index("c")
    base0 = _K + wid * _PER_W

    lanes = lax.iota(jnp.int32, _L)
    pltpu.sync_copy(w_hbm, w_tile)

    def stage(g, b):
        base = base0 + g * _CHUNK
        pltpu.async_copy(x_hbm.at[pl.ds(base * 4, _CHUNK * 4)], xbufs[b],
                         xsems[b])

    stage(0, 0)
    stage(1, 1)

    def body(h, carry):
        for b in range(2):
            g = 2 * h + b
            base = base0 + g * _CHUNK
            pltpu.make_async_copy(
                x_hbm.at[pl.ds(0, _CHUNK * 4)], xbufs[b], xsems[b]).wait()
            for j in range(_G):
                flat = lanes * 4 + (j * _L * 4 + 1)
                idxs[b][pl.ds(j * _L, _L)] = (
                    plsc.load_gather(xbufs[b], [flat]) * D_MODEL)

            @pl.when(g + 2 < _STEPS)
            def _():
                stage(g + 2, b)

            @plsc.parallel_loop(0, _CHUNK, 1, unroll=4)
            def _(r):
                off = pl.multiple_of(idxs[b][pl.ds(r, _L)][0], D_MODEL)
                dst = pl.multiple_of((base - _K + r) * D_MODEL, D_MODEL)
                pltpu.async_copy(
                    w_tile.at[pl.ds(off, D_MODEL)],
                    out_hbm.at[pl.ds(dst, D_MODEL)],
                    wsem)

        return carry

    lax.fori_loop(0, _STEPS // 2, body, 0)

    def drain(i, carry):
        pltpu.make_async_copy(
            out_hbm.at[pl.ds(0, _DRAIN)], dummy, wsem).wait()
        return carry

    lax.fori_loop(0, _NDRAIN, drain, 0)


@jax.jit
def kernel(x, W):
    x2f = x.reshape(_N * 4).astype(jnp.int32)
    x2 = x.reshape(_N, 4).astype(jnp.int32)
    w2 = W.reshape(MINUTE_SIZE * D_MODEL)
    w_pad = jnp.zeros((_V, D_MODEL), jnp.float32).at[:MINUTE_SIZE].set(W)

    mesh = plsc.VectorSubcoreMesh(core_axis_name="c", subcore_axis_name="s")

    def body(x_hbm, w_hbm, out_hbm, w_tile, dummy, xb0, xb1, id0, id1,
             xs0, xs1, ws):
        _sc_kernel(x_hbm, w_hbm, out_hbm, w_tile, dummy,
                   (xb0, xb1), (id0, id1), (xs0, xs1), ws)

    sc_out = pl.kernel(
        body,
        mesh=mesh,
        compiler_params=pltpu.CompilerParams(needs_layout_passes=False),
        out_type=jax.ShapeDtypeStruct((_NSC * D_MODEL,), jnp.float32),
        scratch_types=[
            pltpu.VMEM((MINUTE_SIZE * D_MODEL,), jnp.float32),
            pltpu.VMEM((_DRAIN,), jnp.float32),
            pltpu.VMEM((_CHUNK * 4,), jnp.int32),
            pltpu.VMEM((_CHUNK * 4,), jnp.int32),
            pltpu.VMEM((_CHUNK + _L,), jnp.int32),
            pltpu.VMEM((_CHUNK + _L,), jnp.int32),
            pltpu.SemaphoreType.DMA,
            pltpu.SemaphoreType.DMA,
            pltpu.SemaphoreType.DMA,
        ],
    )(x2f, w2)

    tc_out = pl.pallas_call(
        _tc_body,
        grid=(_GRID,),
        in_specs=[
            pl.BlockSpec((_BLK, 4), lambda i: (i, 0)),
            pl.BlockSpec((_V, D_MODEL), lambda i: (0, 0)),
        ],
        out_specs=pl.BlockSpec((_BLK, D_MODEL), lambda i: (i, 0)),
        out_shape=jax.ShapeDtypeStruct((_K, D_MODEL), jnp.float32),
    )(x2[:_K], w_pad)

    out = jnp.concatenate([tc_out, sc_out.reshape(_NSC, D_MODEL)], axis=0)
    return out.reshape(4096, 200, D_MODEL)


# speedup vs baseline: 1.0733x; 1.0733x over previous
import jax
import jax.numpy as jnp
from jax import lax
from jax.experimental import pallas as pl
from jax.experimental.pallas import tpu as pltpu
from jax.experimental.pallas import tpu_sc as plsc

MINUTE_SIZE = 60
D_MODEL = 512

_N = 4096 * 200
_NW = 32
_PER_W = _N // _NW
_CHUNK = 80
_DIRECT = 40
_MAT = _CHUNK - _DIRECT
_STEPS = _PER_W // _CHUNK
_L = 16
_G = _CHUNK // _L
_DRAIN = 16384
_NDRAIN = _PER_W // _CHUNK * _DIRECT * D_MODEL // _DRAIN


def _sc_kernel(x_hbm, w_hbm, out_hbm, w_tile, dummy, xbufs, idxs, rows,
               xsems, bsems, wsem):
    wid = lax.axis_index("s") * 2 + lax.axis_index("c")
    base0 = wid * _PER_W
    lanes = lax.iota(jnp.int32, _L)

    pltpu.sync_copy(w_hbm, w_tile)

    def stage(g, b):
        base = base0 + g * _CHUNK
        pltpu.async_copy(x_hbm.at[pl.ds(base * 4, _CHUNK * 4)], xbufs[b],
                         xsems[b])

    stage(0, 0)
    stage(1, 1)

    def body(h, carry):
        for b in range(2):
            g = 2 * h + b
            base = base0 + g * _CHUNK
            pltpu.make_async_copy(
                x_hbm.at[pl.ds(0, _CHUNK * 4)], xbufs[b], xsems[b]).wait()
            for j in range(_G):
                flat = lanes * 4 + (j * _L * 4 + 1)
                idxs[b][pl.ds(j * _L, _L)] = (
                    plsc.load_gather(xbufs[b], [flat]) * D_MODEL)

            @pl.when(g + 2 < _STEPS)
            def _():
                stage(g + 2, b)

            @plsc.parallel_loop(0, _DIRECT, 1, unroll=4)
            def _(r):
                off = pl.multiple_of(idxs[b][pl.ds(r, _L)][0], D_MODEL)
                dst = pl.multiple_of((base + r) * D_MODEL, D_MODEL)
                pltpu.async_copy(
                    w_tile.at[pl.ds(off, D_MODEL)],
                    out_hbm.at[pl.ds(dst, D_MODEL)],
                    wsem)

            @pl.when(g >= 2)
            def _():
                pltpu.make_async_copy(
                    rows[b],
                    out_hbm.at[pl.ds(0, _MAT * D_MODEL)],
                    bsems[b]).wait()

            @plsc.parallel_loop(0, _MAT, 1, unroll=4)
            def _(r):
                off = idxs[b][pl.ds(_DIRECT + r, _L)][0]
                for j in range(D_MODEL // _L):
                    rows[b][pl.ds(r * D_MODEL + j * _L, _L)] = (
                        w_tile[pl.ds(off + j * _L, _L)])

            pltpu.async_copy(
                rows[b],
                out_hbm.at[pl.ds((base + _DIRECT) * D_MODEL,
                                 _MAT * D_MODEL)],
                bsems[b])

        return carry

    lax.fori_loop(0, _STEPS // 2, body, 0)

    for b in range(2):
        pltpu.make_async_copy(
            rows[b], out_hbm.at[pl.ds(0, _MAT * D_MODEL)], bsems[b]).wait()

    def drain(i, carry):
        pltpu.make_async_copy(
            out_hbm.at[pl.ds(0, _DRAIN)], dummy, wsem).wait()
        return carry

    lax.fori_loop(0, _NDRAIN, drain, 0)


@jax.jit
def kernel(x, W):
    x2 = x.reshape(_N * 4).astype(jnp.int32)
    w2 = W.reshape(MINUTE_SIZE * D_MODEL)
    mesh = plsc.VectorSubcoreMesh(core_axis_name="c", subcore_axis_name="s")

    def body(x_hbm, w_hbm, out_hbm, w_tile, dummy, xb0, xb1, id0, id1,
             r0, r1, xs0, xs1, bs0, bs1, ws):
        _sc_kernel(x_hbm, w_hbm, out_hbm, w_tile, dummy,
                   (xb0, xb1), (id0, id1), (r0, r1), (xs0, xs1),
                   (bs0, bs1), ws)

    out = pl.kernel(
        body,
        mesh=mesh,
        compiler_params=pltpu.CompilerParams(needs_layout_passes=False),
        out_type=jax.ShapeDtypeStruct((_N * D_MODEL,), jnp.float32),
        scratch_types=[
            pltpu.VMEM((MINUTE_SIZE * D_MODEL,), jnp.float32),
            pltpu.VMEM((_DRAIN,), jnp.float32),
            pltpu.VMEM((_CHUNK * 4,), jnp.int32),
            pltpu.VMEM((_CHUNK * 4,), jnp.int32),
            pltpu.VMEM((_CHUNK + _L,), jnp.int32),
            pltpu.VMEM((_CHUNK + _L,), jnp.int32),
            pltpu.VMEM((_MAT * D_MODEL,), jnp.float32),
            pltpu.VMEM((_MAT * D_MODEL,), jnp.float32),
            pltpu.SemaphoreType.DMA,
            pltpu.SemaphoreType.DMA,
            pltpu.SemaphoreType.DMA,
            pltpu.SemaphoreType.DMA,
            pltpu.SemaphoreType.DMA,
        ],
    )(x2, w2)
    return out.reshape(4096, 200, D_MODEL)


# speedup vs baseline: 1.0762x; 1.0027x over previous
import jax
import jax.numpy as jnp
from jax import lax
from jax.experimental import pallas as pl
from jax.experimental.pallas import tpu as pltpu
from jax.experimental.pallas import tpu_sc as plsc

MINUTE_SIZE = 60
D_MODEL = 512

_N = 4096 * 200
_NW = 32
_PER_W = _N // _NW
_CHUNK = 80
_STEPS = _PER_W // _CHUNK
_L = 16
_G = _CHUNK // _L
_DRAIN = 65536
_NDRAIN = _PER_W * D_MODEL // _DRAIN


def _sc_kernel(x_hbm, w_hbm, out_hbm, w_tile, dummy, xbufs, idxs,
               xsems, wsem):
    wid = lax.axis_index("s") * 2 + lax.axis_index("c")
    base0 = wid * _PER_W
    lanes = lax.iota(jnp.int32, _L)

    pltpu.sync_copy(w_hbm, w_tile)

    def stage(g, b):
        base = base0 + g * _CHUNK
        pltpu.async_copy(x_hbm.at[pl.ds(base * 4, _CHUNK * 4)], xbufs[b],
                         xsems[b])

    stage(0, 0)
    stage(1, 1)

    def body(h, carry):
        for b in range(2):
            g = 2 * h + b
            base = base0 + g * _CHUNK
            pltpu.make_async_copy(
                x_hbm.at[pl.ds(0, _CHUNK * 4)], xbufs[b], xsems[b]).wait()
            for j in range(_G):
                flat = lanes * 4 + (j * _L * 4 + 1)
                idxs[b][pl.ds(j * _L, _L)] = (
                    plsc.load_gather(xbufs[b], [flat]) * D_MODEL)

            @pl.when(g + 2 < _STEPS)
            def _():
                stage(g + 2, b)

            @plsc.parallel_loop(0, _CHUNK, 1, unroll=8)
            def _(r):
                off = pl.multiple_of(idxs[b][pl.ds(r, _L)][0], D_MODEL)
                dst = pl.multiple_of((base + r) * D_MODEL, D_MODEL)
                pltpu.async_copy(
                    w_tile.at[pl.ds(off, D_MODEL)],
                    out_hbm.at[pl.ds(dst, D_MODEL)],
                    wsem)

        return carry

    lax.fori_loop(0, _STEPS // 2, body, 0)

    def drain(i, carry):
        pltpu.make_async_copy(
            out_hbm.at[pl.ds(0, _DRAIN)], dummy, wsem).wait()
        return carry

    lax.fori_loop(0, _NDRAIN, drain, 0)


@jax.jit
def kernel(x, W):
    x2 = x.reshape(_N * 4).astype(jnp.int32)
    w2 = W.reshape(MINUTE_SIZE * D_MODEL)
    mesh = plsc.VectorSubcoreMesh(core_axis_name="c", subcore_axis_name="s")

    def body(x_hbm, w_hbm, out_hbm, w_tile, dummy, xb0, xb1, id0, id1,
             xs0, xs1, ws):
        _sc_kernel(x_hbm, w_hbm, out_hbm, w_tile, dummy,
                   (xb0, xb1), (id0, id1), (xs0, xs1), ws)

    out = pl.kernel(
        body,
        mesh=mesh,
        compiler_params=pltpu.CompilerParams(needs_layout_passes=False),
        out_type=jax.ShapeDtypeStruct((_N * D_MODEL,), jnp.float32),
        scratch_types=[
            pltpu.VMEM((MINUTE_SIZE * D_MODEL,), jnp.float32),
            pltpu.VMEM((_DRAIN,), jnp.float32),
            pltpu.VMEM((_CHUNK * 4,), jnp.int32),
            pltpu.VMEM((_CHUNK * 4,), jnp.int32),
            pltpu.VMEM((_CHUNK + _L,), jnp.int32),
            pltpu.VMEM((_CHUNK + _L,), jnp.int32),
            pltpu.SemaphoreType.DMA,
            pltpu.SemaphoreType.DMA,
            pltpu.SemaphoreType.DMA,
        ],
    )(x2, w2)
    return out.reshape(4096, 200, D_MODEL)
